# baseline (speedup 1.0000x reference)
import functools
import jax
import jax.numpy as jnp
from jax import lax
from jax.experimental import pallas as pl
from jax.experimental.pallas import tpu as pltpu
from jax.experimental.pallas import tpu_sc as plsc

_NC = 2
_NS = 16
_L = 16
_CH = 32


def _sc_add(x, emb):
    B, S, D = x.shape
    nw = _NC * _NS
    rows_per_w = S // nw
    mesh = plsc.VectorSubcoreMesh(core_axis_name="c", subcore_axis_name="s")

    @functools.partial(
        pl.kernel,
        mesh=mesh,
        out_type=jax.ShapeDtypeStruct((B, S, D), jnp.float32),
        scratch_types=[
            pltpu.VMEM((_CH, D), jnp.float32),
            pltpu.VMEM((_CH, D), jnp.float32),
        ],
    )
    def body(x_hbm, emb_hbm, out_hbm, emb_v, x_v):
        wid = lax.axis_index("s") * _NC + lax.axis_index("c")
        base = wid * rows_per_w
        for c in range(rows_per_w // _CH):
            row0 = base + c * _CH
            pltpu.sync_copy(emb_hbm.at[pl.ds(row0, _CH)], emb_v)
            for b in range(B):
                pltpu.sync_copy(x_hbm.at[b, pl.ds(row0, _CH)], x_v)

                def add_body(i, carry):
                    r = i // (D // _L)
                    j = (i % (D // _L)) * _L
                    x_v[r, pl.ds(j, _L)] = (
                        x_v[r, pl.ds(j, _L)] + emb_v[r, pl.ds(j, _L)]
                    )
                    return carry

                lax.fori_loop(0, _CH * D // _L, add_body, 0)
                pltpu.sync_copy(x_v, out_hbm.at[b, pl.ds(row0, _CH)])

    return body(x, emb[:S])


def kernel(x, emb):
    return _sc_add(x, emb)


# speedup vs baseline: 1.5050x; 1.5050x over previous
import functools
import jax
import jax.numpy as jnp
from jax import lax
from jax.experimental import pallas as pl
from jax.experimental.pallas import tpu as pltpu
from jax.experimental.pallas import tpu_sc as plsc

_NC = 2
_NS = 16
_L = 16
_CH = 16


def _sc_add(x, emb):
    B, S, D = x.shape
    nw = _NC * _NS
    rows_per_w = S // nw
    nchunk = rows_per_w // _CH
    nsteps = nchunk * B
    groups = _CH * D // _L
    mesh = plsc.VectorSubcoreMesh(core_axis_name="c", subcore_axis_name="s")

    @functools.partial(
        pl.kernel,
        mesh=mesh,
        out_type=jax.ShapeDtypeStruct((B, S, D), jnp.float32),
        scratch_types=[
            pltpu.VMEM((2, _CH, D), jnp.float32),
            pltpu.VMEM((2, _CH, D), jnp.float32),
            pltpu.SemaphoreType.DMA,
            pltpu.SemaphoreType.DMA,
            pltpu.SemaphoreType.DMA,
            pltpu.SemaphoreType.DMA,
            pltpu.SemaphoreType.DMA,
            pltpu.SemaphoreType.DMA,
        ],
    )
    def body(x_hbm, emb_hbm, out_hbm, x_v, emb_v,
             lx0, lx1, le0, le1, st0, st1):
        lx = (lx0, lx1)
        le = (le0, le1)
        st = (st0, st1)
        wid = lax.axis_index("s") * _NC + lax.axis_index("c")
        base = wid * rows_per_w

        def x_load(t):
            c, b = divmod(t, B)
            buf = t % 2
            return pltpu.async_copy(
                x_hbm.at[b, pl.ds(base + c * _CH, _CH)],
                x_v.at[buf], lx[buf])

        def emb_load(c):
            buf = c % 2
            return pltpu.async_copy(
                emb_hbm.at[pl.ds(base + c * _CH, _CH)],
                emb_v.at[buf], le[buf])

        def x_store(t):
            c, b = divmod(t, B)
            buf = t % 2
            return pltpu.async_copy(
                x_v.at[buf],
                out_hbm.at[b, pl.ds(base + c * _CH, _CH)], st[buf])

        emb_load(0).wait()
        h_x = {0: x_load(0)}
        h_st = {}

        for t in range(nsteps):
            c, b = divmod(t, B)
            buf = t % 2
            h_x.pop(t).wait()
            if b == 0 and c > 0:
                pass
            if t + 1 < nsteps:
                if t - 1 >= 0:
                    h_st.pop(t - 1).wait()
                h_x[t + 1] = x_load(t + 1)
            if b == 0 and c + 1 < nchunk:
                h_e = emb_load(c + 1)
            if b == B - 1 and c + 1 < nchunk:
                h_e.wait()

            ebuf = c % 2

            def add_body(i, carry):
                r = i // (D // _L)
                j = (i % (D // _L)) * _L
                plsc.addupdate(
                    x_v.at[buf, r, pl.ds(j, _L)],
                    emb_v[ebuf, r, pl.ds(j, _L)])
                return carry

            lax.fori_loop(0, groups, add_body, 0, unroll=8)
            h_st[t] = x_store(t)

        h_st.pop(nsteps - 2).wait()
        h_st.pop(nsteps - 1).wait()

    return body(x, emb[:S])


def kernel(x, emb):
    return _sc_add(x, emb)


# speedup vs baseline: 2.5531x; 1.6964x over previous
import functools
import jax
import jax.numpy as jnp
from jax import lax
from jax.experimental import pallas as pl
from jax.experimental.pallas import tpu as pltpu
from jax.experimental.pallas import tpu_sc as plsc

_NC = 2
_NS = 16
_L = 16
_CH = 16


def _sc_add(x, emb):
    B, S, D = x.shape
    nw = _NC * _NS
    rows_per_w = S // nw
    nchunk = rows_per_w // _CH
    nsteps = nchunk * B
    groups = _CH * D // _L
    mesh = plsc.VectorSubcoreMesh(core_axis_name="c", subcore_axis_name="s")

    @functools.partial(
        pl.kernel,
        mesh=mesh,
        out_type=jax.ShapeDtypeStruct((B, S, D), jnp.float32),
        scratch_types=[
            pltpu.VMEM((2, _CH, D), jnp.float32),
            pltpu.VMEM((2, _CH, D), jnp.float32),
            pltpu.SemaphoreType.DMA,
            pltpu.SemaphoreType.DMA,
            pltpu.SemaphoreType.DMA,
            pltpu.SemaphoreType.DMA,
            pltpu.SemaphoreType.DMA,
            pltpu.SemaphoreType.DMA,
        ],
    )
    def body(x_hbm, emb_hbm, out_hbm, x_v, emb_v,
             lx0, lx1, le0, le1, st0, st1):
        lx = (lx0, lx1)
        le = (le0, le1)
        st = (st0, st1)
        wid = lax.axis_index("s") * _NC + lax.axis_index("c")
        base = wid * rows_per_w

        def x_load(t):
            c, b = divmod(t, B)
            buf = t % 2
            return pltpu.async_copy(
                x_hbm.at[b, pl.ds(base + c * _CH, _CH)],
                x_v.at[buf], lx[buf])

        def emb_load(c):
            buf = c % 2
            return pltpu.async_copy(
                emb_hbm.at[pl.ds(base + c * _CH, _CH)],
                emb_v.at[buf], le[buf])

        def x_store(t):
            c, b = divmod(t, B)
            buf = t % 2
            return pltpu.async_copy(
                x_v.at[buf],
                out_hbm.at[b, pl.ds(base + c * _CH, _CH)], st[buf])

        emb_load(0).wait()
        h_x = {0: x_load(0)}
        h_st = {}

        for t in range(nsteps):
            c, b = divmod(t, B)
            buf = t % 2
            h_x.pop(t).wait()
            if b == 0 and c > 0:
                pass
            if t + 1 < nsteps:
                if t - 1 >= 0:
                    h_st.pop(t - 1).wait()
                h_x[t + 1] = x_load(t + 1)
            if b == 0 and c + 1 < nchunk:
                h_e = emb_load(c + 1)
            if b == B - 1 and c + 1 < nchunk:
                h_e.wait()

            ebuf = c % 2

            @plsc.parallel_loop(0, groups, unroll=8)
            def _(i):
                r = i // (D // _L)
                j = (i % (D // _L)) * _L
                plsc.addupdate(
                    x_v.at[buf, r, pl.ds(j, _L)],
                    emb_v[ebuf, r, pl.ds(j, _L)])
            h_st[t] = x_store(t)

        h_st.pop(nsteps - 2).wait()
        h_st.pop(nsteps - 1).wait()

    return body(x, emb[:S])


def kernel(x, emb):
    return _sc_add(x, emb)


# speedup vs baseline: 2.8220x; 1.1053x over previous
import functools
import jax
import jax.numpy as jnp
from jax import lax
from jax.experimental import pallas as pl
from jax.experimental.pallas import tpu as pltpu
from jax.experimental.pallas import tpu_sc as plsc

_NC = 2
_NS = 16
_L = 16
_CH = 16
_NBUF = 4


def _sc_add(x, emb):
    B, S, D = x.shape
    nw = _NC * _NS
    rows_per_w = S // nw
    nchunk = rows_per_w // _CH
    nsteps = nchunk * B
    groups = _CH * D // _L
    mesh = plsc.VectorSubcoreMesh(core_axis_name="c", subcore_axis_name="s")

    @functools.partial(
        pl.kernel,
        mesh=mesh,
        out_type=jax.ShapeDtypeStruct((B, S, D), jnp.float32),
        scratch_types=[
            pltpu.VMEM((_NBUF, _CH, D), jnp.float32),
            pltpu.VMEM((2, _CH, D), jnp.float32),
        ]
        + [pltpu.SemaphoreType.DMA] * _NBUF
        + [pltpu.SemaphoreType.DMA] * 2
        + [pltpu.SemaphoreType.DMA] * _NBUF,
    )
    def body(x_hbm, emb_hbm, out_hbm, x_v, emb_v, *sems):
        lx = sems[:_NBUF]
        le = sems[_NBUF:_NBUF + 2]
        st = sems[_NBUF + 2:]
        wid = lax.axis_index("s") * _NC + lax.axis_index("c")
        base = wid * rows_per_w

        def x_load(t):
            c, b = divmod(t, B)
            buf = t % _NBUF
            return pltpu.async_copy(
                x_hbm.at[b, pl.ds(base + c * _CH, _CH)],
                x_v.at[buf], lx[buf])

        def emb_load(c):
            buf = c % 2
            return pltpu.async_copy(
                emb_hbm.at[pl.ds(base + c * _CH, _CH)],
                emb_v.at[buf], le[buf])

        def x_store(t):
            c, b = divmod(t, B)
            buf = t % _NBUF
            return pltpu.async_copy(
                x_v.at[buf],
                out_hbm.at[b, pl.ds(base + c * _CH, _CH)], st[buf])

        h_e = emb_load(0)
        h_x = {t: x_load(t) for t in range(min(_NBUF - 1, nsteps))}
        h_e.wait()
        h_st = {}

        for t in range(nsteps):
            c, b = divmod(t, B)
            buf = t % _NBUF
            h_x.pop(t).wait()
            if b == 0 and c + 1 < nchunk:
                h_e = emb_load(c + 1)
            if b == B - 1 and c + 1 < nchunk:
                h_e.wait()

            ebuf = c % 2

            @plsc.parallel_loop(0, groups, unroll=16)
            def _(i):
                r = i // (D // _L)
                j = (i % (D // _L)) * _L
                plsc.addupdate(
                    x_v.at[buf, r, pl.ds(j, _L)],
                    emb_v[ebuf, r, pl.ds(j, _L)])

            h_st[t] = x_store(t)
            u = t + _NBUF - 1
            if u < nsteps:
                if u - _NBUF >= 0:
                    h_st.pop(u - _NBUF).wait()
                h_x[u] = x_load(u)

        for t in sorted(h_st):
            h_st.pop(t).wait()

    return body(x, emb[:S])


def kernel(x, emb):
    return _sc_add(x, emb)


# speedup vs baseline: 3.0974x; 1.0976x over previous
import functools
import jax
import jax.numpy as jnp
from jax import lax
from jax.experimental import pallas as pl
from jax.experimental.pallas import tpu as pltpu
from jax.experimental.pallas import tpu_sc as plsc

_NC = 2
_NS = 16
_L = 16
_CH = 16
_NBUF = 4


def _sc_add(x, emb):
    B, S, D = x.shape
    nw = _NC * _NS
    rows_per_w = S // nw
    nchunk = rows_per_w // _CH
    nsteps = nchunk * B
    groups = _CH * D // _L
    mesh = plsc.VectorSubcoreMesh(core_axis_name="c", subcore_axis_name="s")

    @functools.partial(
        pl.kernel,
        mesh=mesh,
        out_type=jax.ShapeDtypeStruct((B, S, D), jnp.float32),
        scratch_types=[
            pltpu.VMEM((_NBUF, _CH, D), jnp.float32),
            pltpu.VMEM((2, _CH, D), jnp.float32),
        ]
        + [pltpu.SemaphoreType.DMA] * _NBUF
        + [pltpu.SemaphoreType.DMA] * 2
        + [pltpu.SemaphoreType.DMA] * _NBUF,
    )
    def body(x_hbm, emb_hbm, out_hbm, x_v, emb_v, *sems):
        lx = sems[:_NBUF]
        le = sems[_NBUF:_NBUF + 2]
        st = sems[_NBUF + 2:]
        wid = lax.axis_index("s") * _NC + lax.axis_index("c")
        base = wid * rows_per_w

        def x_load(t):
            c, b = divmod(t, B)
            buf = t % _NBUF
            return pltpu.async_copy(
                x_hbm.at[b, pl.ds(base + c * _CH, _CH)],
                x_v.at[buf], lx[buf])

        def emb_load(c):
            buf = c % 2
            return pltpu.async_copy(
                emb_hbm.at[pl.ds(base + c * _CH, _CH)],
                emb_v.at[buf], le[buf])

        def x_store(t):
            c, b = divmod(t, B)
            buf = t % _NBUF
            return pltpu.async_copy(
                x_v.at[buf],
                out_hbm.at[b, pl.ds(base + c * _CH, _CH)], st[buf])

        h_e = emb_load(0)
        h_x = {t: x_load(t) for t in range(min(_NBUF - 1, nsteps))}
        h_e.wait()
        h_st = {}

        for t in range(nsteps):
            c, b = divmod(t, B)
            buf = t % _NBUF
            h_x.pop(t).wait()
            if b == 0 and c + 1 < nchunk:
                h_e = emb_load(c + 1)
            if b == B - 1 and c + 1 < nchunk:
                h_e.wait()

            ebuf = c % 2

            @plsc.parallel_loop(0, groups, unroll=16)
            def _(i):
                r = i // (D // _L)
                j = (i % (D // _L)) * _L
                plsc.addupdate(
                    x_v.at[buf, r, pl.ds(j, _L)],
                    emb_v[ebuf, r, pl.ds(j, _L)])

            h_st[t] = x_store(t)
            u = t + _NBUF - 1
            if u < nsteps:
                if u - _NBUF >= 0:
                    h_st.pop(u - _NBUF).wait()
                h_x[u] = x_load(u)

        for t in sorted(h_st):
            h_st.pop(t).wait()

    return body(x, emb)


def kernel(x, emb):
    return _sc_add(x, emb)


# speedup vs baseline: 3.0992x; 1.0006x over previous
import functools
import jax
import jax.numpy as jnp
from jax import lax
from jax.experimental import pallas as pl
from jax.experimental.pallas import tpu as pltpu
from jax.experimental.pallas import tpu_sc as plsc

_NC = 2
_NS = 16
_L = 16
_CH = 16
_NBUF = 5


def _sc_add(x, emb):
    B, S, D = x.shape
    nw = _NC * _NS
    rows_per_w = S // nw
    nchunk = rows_per_w // _CH
    nsteps = nchunk * B
    groups = _CH * D // _L
    mesh = plsc.VectorSubcoreMesh(core_axis_name="c", subcore_axis_name="s")

    @functools.partial(
        pl.kernel,
        mesh=mesh,
        out_type=jax.ShapeDtypeStruct((B, S, D), jnp.float32),
        scratch_types=[
            pltpu.VMEM((_NBUF, _CH, D), jnp.float32),
            pltpu.VMEM((2, _CH, D), jnp.float32),
        ]
        + [pltpu.SemaphoreType.DMA] * _NBUF
        + [pltpu.SemaphoreType.DMA] * 2
        + [pltpu.SemaphoreType.DMA] * _NBUF,
    )
    def body(x_hbm, emb_hbm, out_hbm, x_v, emb_v, *sems):
        lx = sems[:_NBUF]
        le = sems[_NBUF:_NBUF + 2]
        st = sems[_NBUF + 2:]
        wid = lax.axis_index("s") * _NC + lax.axis_index("c")
        base = wid * rows_per_w

        def x_load(t):
            c, b = divmod(t, B)
            buf = t % _NBUF
            return pltpu.async_copy(
                x_hbm.at[b, pl.ds(base + c * _CH, _CH)],
                x_v.at[buf], lx[buf])

        def emb_load(c):
            buf = c % 2
            return pltpu.async_copy(
                emb_hbm.at[pl.ds(base + c * _CH, _CH)],
                emb_v.at[buf], le[buf])

        def x_store(t):
            c, b = divmod(t, B)
            buf = t % _NBUF
            return pltpu.async_copy(
                x_v.at[buf],
                out_hbm.at[b, pl.ds(base + c * _CH, _CH)], st[buf])

        h_e = emb_load(0)
        h_x = {t: x_load(t) for t in range(min(_NBUF - 1, nsteps))}
        h_e.wait()
        h_st = {}

        for t in range(nsteps):
            c, b = divmod(t, B)
            buf = t % _NBUF
            h_x.pop(t).wait()
            if b == 0 and c + 1 < nchunk:
                h_e = emb_load(c + 1)
            if b == B - 1 and c + 1 < nchunk:
                h_e.wait()

            ebuf = c % 2

            @plsc.parallel_loop(0, groups, unroll=16)
            def _(i):
                r = i // (D // _L)
                j = (i % (D // _L)) * _L
                plsc.addupdate(
                    x_v.at[buf, r, pl.ds(j, _L)],
                    emb_v[ebuf, r, pl.ds(j, _L)])

            h_st[t] = x_store(t)
            u = t + _NBUF - 1
            if u < nsteps:
                if u - _NBUF >= 0:
                    h_st.pop(u - _NBUF).wait()
                h_x[u] = x_load(u)

        for t in sorted(h_st):
            h_st.pop(t).wait()

    return body(x, emb)


def kernel(x, emb):
    return _sc_add(x, emb)
